# baseline (speedup 1.0000x reference)
import jax
import jax.numpy as jnp
from jax import lax
from jax.experimental import pallas as pl
from jax.experimental.pallas import tpu as pltpu
from jax.experimental.pallas import tpu_sc as plsc

NC, NS, L = 2, 16, 16
NW = NC * NS
F32_EPS = 1.1920929e-07


def _logits_body(x_ref, wg_ref, gnT_ref, logT_ref, noisyT_ref):
    lg = lax.dot_general(wg_ref[...], x_ref[...], (((1,), (1,)), ((), ())),
                         preferred_element_type=jnp.float32)
    logT_ref[...] = lg
    noisyT_ref[...] = lg + gnT_ref[...]


def _make_gate(E, S, C):
    slots = E * C
    tpt = S // NS
    n_loc = tpt // L

    def body(logT, noisyT,
             src_hbm, dst1_hbm, dst2_hbm, w1_hbm, w2_hbm,
             lg_v, ns_v, e1l, e2l, g1l, g2l, r1l, r2l,
             cnts_v, allc_v, tot_v, off_v,
             d1m_v, d2m_v, d1o_v, d2o_v, w1o_v, w2o_v,
             d1a_v, d2a_v, src_v,
             cnts_sh, d1_sh, d2_sh):
        cid = lax.axis_index("c")
        sid = lax.axis_index("s")
        wid = sid * NC + cid
        s0 = sid * tpt
        iota = lax.iota(jnp.int32, L)

        pltpu.sync_copy(logT, lg_v)
        pltpu.sync_copy(noisyT, ns_v)

        c1 = [jnp.int32(0)] * E
        c2 = [jnp.int32(0)] * E
        for c in range(n_loc):
            ds = pl.ds(s0 + c * L, L)
            dsl = pl.ds(c * L, L)
            ls = [lg_v[e, ds] for e in range(E)]
            m = ls[0]
            for e in range(1, E):
                m = jnp.maximum(m, ls[e])
            exs = [jnp.exp(ls[e] - m) for e in range(E)]
            Z = exs[0]
            for e in range(1, E):
                Z = Z + exs[e]
            idx1 = jnp.full((L,), 127, jnp.int32)
            for e in range(E):
                idx1 = jnp.minimum(idx1, jnp.where(ls[e] == m, e, 127))
            g1 = jnp.zeros((L,), jnp.float32)
            for e in range(E):
                g1 = jnp.where(idx1 == e, exs[e], g1)
            neg = jnp.full((L,), -jnp.inf, jnp.float32)
            nss = []
            m2 = neg
            for e in range(E):
                nv = jnp.where(idx1 == e, neg, ns_v[e, ds])
                nss.append(nv)
                m2 = jnp.maximum(m2, nv)
            idx2 = jnp.full((L,), 127, jnp.int32)
            for e in range(E):
                idx2 = jnp.minimum(idx2, jnp.where(nss[e] == m2, e, 127))
            g2 = jnp.zeros((L,), jnp.float32)
            for e in range(E):
                g2 = jnp.where(idx2 == e, exs[e], g2)
            rZ = 1.0 / Z
            e1l[dsl] = idx1
            e2l[dsl] = idx2
            g1l[dsl] = g1 * rZ
            g2l[dsl] = g2 * rZ
            r1 = jnp.zeros((L,), jnp.int32)
            r2 = jnp.zeros((L,), jnp.int32)
            for e in range(E):
                mask = idx1 == e
                mi = jnp.where(mask, 1, 0).astype(jnp.int32)
                pc = plsc.cumsum(mi)
                r1 = jnp.where(mask, c1[e] + pc - 1, r1)
                c1[e] = c1[e] + jnp.sum(mi)
                mask = idx2 == e
                mi = jnp.where(mask, 1, 0).astype(jnp.int32)
                pc = plsc.cumsum(mi)
                r2 = jnp.where(mask, c2[e] + pc - 1, r2)
                c2[e] = c2[e] + jnp.sum(mi)
            r1l[dsl] = r1
            r2l[dsl] = r2

        cv = jnp.zeros((L,), jnp.int32)
        for e in range(E):
            cv = jnp.where(iota == e, c1[e], cv)
            cv = jnp.where(iota == (E + e), c2[e], cv)
        cnts_v[...] = cv
        pltpu.sync_copy(cnts_v, cnts_sh.at[sid])
        plsc.subcore_barrier()

        pltpu.sync_copy(cnts_sh, allc_v)
        sv = jnp.broadcast_to(sid, (L,)).astype(jnp.int32)
        pref = jnp.zeros((L,), jnp.int32)
        tot = jnp.zeros((L,), jnp.int32)
        for t in range(NS):
            row = allc_v[t]
            tv = jnp.full((L,), t, jnp.int32)
            pref = pref + jnp.where(tv < sv, row, 0)
            tot = tot + row
        tot_v[...] = tot
        sh = plsc.load_gather(tot_v, [jnp.maximum(iota - E, 0)])
        off = pref + jnp.where(iota >= E, sh, 0)
        off_v[...] = off

        for c in range(n_loc):
            dsl = pl.ds(c * L, L)
            idx1 = e1l[dsl]
            idx2 = e2l[dsl]
            base1 = plsc.load_gather(off_v, [idx1])
            base2 = plsc.load_gather(off_v, [idx2 + E])
            l1 = base1 + r1l[dsl]
            l2 = base2 + r2l[dsl]
            k1 = l1 < C
            k2 = l2 < C
            g1k = jnp.where(k1, g1l[dsl], 0.0)
            g2k = jnp.where(k2, g2l[dsl], 0.0)
            rd = 1.0 / jnp.maximum(g1k + g2k, F32_EPS)
            d1 = idx1 * C + l1
            d2 = idx2 * C + l2
            d1m_v[dsl] = jnp.where(k1, d1, slots)
            d2m_v[dsl] = jnp.where(k2, d2, slots)
            d1o_v[dsl] = jnp.where(k1, d1, 0)
            d2o_v[dsl] = jnp.where(k2, d2, 0)
            w1o_v[dsl] = g1k * rd
            w2o_v[dsl] = g2k * rd

        @pl.when(cid == 0)
        def _():
            pltpu.sync_copy(d1o_v, dst1_hbm.at[pl.ds(s0, tpt)])
            pltpu.sync_copy(d2o_v, dst2_hbm.at[pl.ds(s0, tpt)])
            pltpu.sync_copy(w1o_v, w1_hbm.at[pl.ds(s0, tpt)])
            pltpu.sync_copy(w2o_v, w2_hbm.at[pl.ds(s0, tpt)])

        pltpu.sync_copy(d1m_v, d1_sh.at[pl.ds(s0, tpt)])
        pltpu.sync_copy(d2m_v, d2_sh.at[pl.ds(s0, tpt)])
        plsc.subcore_barrier()

        pltpu.sync_copy(d1_sh, d1a_v)
        pltpu.sync_copy(d2_sh, d2a_v)

        def zinit(i, carry):
            src_v[pl.ds(pl.multiple_of(i * L, L), L)] = jnp.zeros(
                (L,), jnp.int32)
            return carry

        lax.fori_loop(0, slots // L, zinit, 0)

        def rebuild(i, carry):
            ds = pl.ds(pl.multiple_of(i * L, L), L)
            tid = (i * L + iota).astype(jnp.int32)
            d1c = d1a_v[ds]
            plsc.store_scatter(src_v, [d1c], tid, mask=d1c < slots)
            d2c = d2a_v[ds]
            plsc.store_scatter(src_v, [d2c], tid, mask=d2c < slots)
            return carry

        lax.fori_loop(0, S // L, rebuild, 0)

        spt = slots // NS
        @pl.when(cid == 0)
        def _():
            pltpu.sync_copy(src_v.at[pl.ds(sid * spt, spt)],
                            src_hbm.at[pl.ds(sid * spt, spt)])

    mesh = plsc.VectorSubcoreMesh(core_axis_name="c", subcore_axis_name="s")
    return pl.kernel(
        body,
        out_type=[
            jax.ShapeDtypeStruct((slots,), jnp.int32),
            jax.ShapeDtypeStruct((S,), jnp.int32),
            jax.ShapeDtypeStruct((S,), jnp.int32),
            jax.ShapeDtypeStruct((S,), jnp.float32),
            jax.ShapeDtypeStruct((S,), jnp.float32),
        ],
        mesh=mesh,
        compiler_params=pltpu.CompilerParams(needs_layout_passes=False),
        scratch_types=[
            pltpu.VMEM((E, S), jnp.float32),
            pltpu.VMEM((E, S), jnp.float32),
            pltpu.VMEM((tpt,), jnp.int32),
            pltpu.VMEM((tpt,), jnp.int32),
            pltpu.VMEM((tpt,), jnp.float32),
            pltpu.VMEM((tpt,), jnp.float32),
            pltpu.VMEM((tpt,), jnp.int32),
            pltpu.VMEM((tpt,), jnp.int32),
            pltpu.VMEM((L,), jnp.int32),
            pltpu.VMEM((NS, L), jnp.int32),
            pltpu.VMEM((L,), jnp.int32),
            pltpu.VMEM((L,), jnp.int32),
            pltpu.VMEM((tpt,), jnp.int32),
            pltpu.VMEM((tpt,), jnp.int32),
            pltpu.VMEM((tpt,), jnp.int32),
            pltpu.VMEM((tpt,), jnp.int32),
            pltpu.VMEM((tpt,), jnp.float32),
            pltpu.VMEM((tpt,), jnp.float32),
            pltpu.VMEM((S,), jnp.int32),
            pltpu.VMEM((S,), jnp.int32),
            pltpu.VMEM((slots,), jnp.int32),
            pltpu.VMEM_SHARED((NS, L), jnp.int32),
            pltpu.VMEM_SHARED((S,), jnp.int32),
            pltpu.VMEM_SHARED((S,), jnp.int32),
        ],
    )


def _make_dispatch_group(S, M, gslots, g):
    spw = gslots // NW

    def body(x_ref, src_ref, disp_ref, idx_v, rows_v, sem):
        wid = lax.axis_index("s") * NC + lax.axis_index("c")
        base = wid * spw
        pltpu.sync_copy(src_ref.at[pl.ds(g * gslots + base, spw)], idx_v)
        pltpu.async_copy(x_ref.at[idx_v], rows_v, sem).wait()
        pltpu.sync_copy(rows_v, disp_ref.at[pl.ds(base, spw)])

    mesh = plsc.VectorSubcoreMesh(core_axis_name="c", subcore_axis_name="s")
    return pl.kernel(
        body,
        out_type=jax.ShapeDtypeStruct((gslots, M), jnp.float32),
        mesh=mesh,
        compiler_params=pltpu.CompilerParams(needs_layout_passes=False),
        scratch_types=[
            pltpu.VMEM((spw,), jnp.int32),
            pltpu.VMEM((spw, M), jnp.float32),
            pltpu.SemaphoreType.DMA,
        ],
    )


def _expert_body(disp_ref, We_ref, be_ref, eo_ref):
    acc = lax.dot_general(disp_ref[...].astype(jnp.bfloat16),
                          We_ref[0].astype(jnp.bfloat16),
                          (((1,), (1,)), ((), ())),
                          preferred_element_type=jnp.float32)
    eo_ref[...] = acc + be_ref[0]


def _expert_body_alias(eo_in_ref, disp_ref, We_ref, be_ref, eo_ref):
    acc = lax.dot_general(disp_ref[...].astype(jnp.bfloat16),
                          We_ref[0].astype(jnp.bfloat16),
                          (((1,), (1,)), ((), ())),
                          preferred_element_type=jnp.float32)
    eo_ref[...] = acc + be_ref[0]


def _make_combine(S, C, E, Dout):
    per_w = S // NW
    tchunk = 16
    nch = per_w // tchunk

    def body(eo_ref, dst1_ref, dst2_ref, w1_ref, w2_ref, y_ref,
             d1_v, d2_v, w1_v, w2_v,
             r1a, r1b, r2a, r2b, outa, outb,
             sg1a, sg1b, sg2a, sg2b, swa, swb):
        wid = lax.axis_index("s") * NC + lax.axis_index("c")
        base = wid * per_w
        pltpu.sync_copy(dst1_ref.at[pl.ds(base, per_w)], d1_v)
        pltpu.sync_copy(dst2_ref.at[pl.ds(base, per_w)], d2_v)
        pltpu.sync_copy(w1_ref.at[pl.ds(base, per_w)], w1_v)
        pltpu.sync_copy(w2_ref.at[pl.ds(base, per_w)], w2_v)
        r1s = [r1a, r1b]
        r2s = [r2a, r2b]
        outs = [outa, outb]
        sg1 = [sg1a, sg1b]
        sg2 = [sg2a, sg2b]
        sw = [swa, swb]

        def gath(h):
            b = h % 2
            c1 = pltpu.async_copy(
                eo_ref.at[d1_v.at[pl.ds(h * tchunk, tchunk)]], r1s[b], sg1[b])
            c2 = pltpu.async_copy(
                eo_ref.at[d2_v.at[pl.ds(h * tchunk, tchunk)]], r2s[b], sg2[b])
            return (c1, c2)

        gs = [gath(0), gath(1)]
        ws = [None, None]
        for h in range(nch):
            b = h % 2
            gs[b][0].wait()
            gs[b][1].wait()
            if h >= 2:
                ws[b].wait()

            def tok(t, carry):
                widx = jnp.broadcast_to(h * tchunk + t, (L,)).astype(jnp.int32)
                w1b = plsc.load_gather(w1_v, [widx])
                w2b = plsc.load_gather(w2_v, [widx])
                for k in range(Dout // L):
                    dsk = pl.ds(k * L, L)
                    outs[b][t, dsk] = (r1s[b][t, dsk] * w1b +
                                       r2s[b][t, dsk] * w2b)
                return carry

            lax.fori_loop(0, tchunk, tok, 0)
            ws[b] = pltpu.async_copy(
                outs[b], y_ref.at[pl.ds(base + h * tchunk, tchunk)], sw[b])
            if h + 2 < nch:
                gs[b] = gath(h + 2)
        ws[(nch - 2) % 2].wait()
        ws[(nch - 1) % 2].wait()

    mesh = plsc.VectorSubcoreMesh(core_axis_name="c", subcore_axis_name="s")
    return pl.kernel(
        body,
        out_type=jax.ShapeDtypeStruct((S, Dout), jnp.float32),
        mesh=mesh,
        compiler_params=pltpu.CompilerParams(needs_layout_passes=False),
        scratch_types=[
            pltpu.VMEM((per_w,), jnp.int32),
            pltpu.VMEM((per_w,), jnp.int32),
            pltpu.VMEM((per_w,), jnp.float32),
            pltpu.VMEM((per_w,), jnp.float32),
            pltpu.VMEM((tchunk, Dout), jnp.float32),
            pltpu.VMEM((tchunk, Dout), jnp.float32),
            pltpu.VMEM((tchunk, Dout), jnp.float32),
            pltpu.VMEM((tchunk, Dout), jnp.float32),
            pltpu.VMEM((tchunk, Dout), jnp.float32),
            pltpu.VMEM((tchunk, Dout), jnp.float32),
            pltpu.SemaphoreType.DMA,
            pltpu.SemaphoreType.DMA,
            pltpu.SemaphoreType.DMA,
            pltpu.SemaphoreType.DMA,
            pltpu.SemaphoreType.DMA,
            pltpu.SemaphoreType.DMA,
        ],
    )


@jax.jit
def kernel(x, wg, We, be, gnoise):
    seq, tok, M = x.shape
    S = seq * tok
    E = wg.shape[0]
    Dout = We.shape[1]
    C = 2 * S // E

    xf = x.reshape(S, M)
    gnT = gnoise.T

    logT, noisyT = pl.pallas_call(
        _logits_body,
        out_shape=[jax.ShapeDtypeStruct((E, S), jnp.float32),
                   jax.ShapeDtypeStruct((E, S), jnp.float32)],
    )(xf, wg, gnT)

    src, dst1, dst2, w1, w2 = _make_gate(E, S, C)(logT, noisyT)

    G = 4
    epg = E // G
    gslots = epg * C
    be3 = be.reshape(E, 1, Dout)
    eo = None
    for g in range(G):
        disp_g = _make_dispatch_group(S, M, gslots, g)(xf, src)
        if g == 0:
            eo = pl.pallas_call(
                _expert_body,
                grid=(epg,),
                in_specs=[
                    pl.BlockSpec((C, M), lambda e: (e, 0)),
                    pl.BlockSpec((1, Dout, M), lambda e: (e, 0, 0)),
                    pl.BlockSpec((1, 1, Dout), lambda e: (e, 0, 0)),
                ],
                out_specs=pl.BlockSpec((C, Dout), lambda e: (e, 0)),
                out_shape=jax.ShapeDtypeStruct((E * C, Dout), jnp.float32),
            )(disp_g, We[:epg], be3[:epg])
        else:
            eo = pl.pallas_call(
                _expert_body_alias,
                grid=(epg,),
                in_specs=[
                    pl.BlockSpec(memory_space=pl.ANY),
                    pl.BlockSpec((C, M), lambda e: (e, 0)),
                    pl.BlockSpec((1, Dout, M), lambda e: (e, 0, 0)),
                    pl.BlockSpec((1, 1, Dout), lambda e: (e, 0, 0)),
                ],
                out_specs=pl.BlockSpec((C, Dout),
                                       lambda e, g=g: (g * epg + e, 0)),
                out_shape=jax.ShapeDtypeStruct((E * C, Dout), jnp.float32),
                input_output_aliases={0: 0},
            )(eo, disp_g, We[g * epg:(g + 1) * epg],
              be3[g * epg:(g + 1) * epg])

    y = _make_combine(S, C, E, Dout)(eo, dst1, dst2, w1, w2)
    return y.reshape(x.shape)


# speedup vs baseline: 1.0806x; 1.0806x over previous
import jax
import jax.numpy as jnp
from jax import lax
from jax.experimental import pallas as pl
from jax.experimental.pallas import tpu as pltpu
from jax.experimental.pallas import tpu_sc as plsc

NC, NS, L = 2, 16, 16
NW = NC * NS
F32_EPS = 1.1920929e-07


def _logits_body(x_ref, wg_ref, gnT_ref, logT_ref, noisyT_ref):
    lg = lax.dot_general(wg_ref[...], x_ref[...], (((1,), (1,)), ((), ())),
                         preferred_element_type=jnp.float32)
    logT_ref[...] = lg
    noisyT_ref[...] = lg + gnT_ref[...]


def _make_gate_dispatch(E, S, C, M):
    slots = E * C
    tpt = S // NS
    spw = slots // NW
    n_loc = tpt // L
    rows_chunk = 32

    def body(logT, noisyT, x_ref,
             disp_ref, dst1_hbm, dst2_hbm, w1_hbm, w2_hbm,
             lg_v, ns_v, e1l, e2l, g1l, g2l, r1l, r2l,
             cnts_v, allc_v, tot_v, off_v,
             d1m_v, d2m_v, d1o_v, d2o_v, w1o_v, w2o_v,
             d1a_v, d2a_v, src_v, rows_v, rows2_v,
             cnts_sh, d1_sh, d2_sh, semr0, semr1, semw0, semw1):
        cid = lax.axis_index("c")
        sid = lax.axis_index("s")
        wid = sid * NC + cid
        s0 = sid * tpt
        iota = lax.iota(jnp.int32, L)

        pltpu.sync_copy(logT, lg_v)
        pltpu.sync_copy(noisyT, ns_v)

        c1 = [jnp.int32(0)] * E
        c2 = [jnp.int32(0)] * E
        for c in range(n_loc):
            ds = pl.ds(s0 + c * L, L)
            dsl = pl.ds(c * L, L)
            ls = [lg_v[e, ds] for e in range(E)]
            m = ls[0]
            for e in range(1, E):
                m = jnp.maximum(m, ls[e])
            exs = [jnp.exp(ls[e] - m) for e in range(E)]
            Z = exs[0]
            for e in range(1, E):
                Z = Z + exs[e]
            idx1 = jnp.full((L,), 127, jnp.int32)
            for e in range(E):
                idx1 = jnp.minimum(idx1, jnp.where(ls[e] == m, e, 127))
            g1 = jnp.zeros((L,), jnp.float32)
            for e in range(E):
                g1 = jnp.where(idx1 == e, exs[e], g1)
            neg = jnp.full((L,), -jnp.inf, jnp.float32)
            nss = []
            m2 = neg
            for e in range(E):
                nv = jnp.where(idx1 == e, neg, ns_v[e, ds])
                nss.append(nv)
                m2 = jnp.maximum(m2, nv)
            idx2 = jnp.full((L,), 127, jnp.int32)
            for e in range(E):
                idx2 = jnp.minimum(idx2, jnp.where(nss[e] == m2, e, 127))
            g2 = jnp.zeros((L,), jnp.float32)
            for e in range(E):
                g2 = jnp.where(idx2 == e, exs[e], g2)
            rZ = 1.0 / Z
            e1l[dsl] = idx1
            e2l[dsl] = idx2
            g1l[dsl] = g1 * rZ
            g2l[dsl] = g2 * rZ
            r1 = jnp.zeros((L,), jnp.int32)
            r2 = jnp.zeros((L,), jnp.int32)
            for e in range(E):
                mask = idx1 == e
                mi = jnp.where(mask, 1, 0).astype(jnp.int32)
                pc = plsc.cumsum(mi)
                r1 = jnp.where(mask, c1[e] + pc - 1, r1)
                c1[e] = c1[e] + jnp.sum(mi)
                mask = idx2 == e
                mi = jnp.where(mask, 1, 0).astype(jnp.int32)
                pc = plsc.cumsum(mi)
                r2 = jnp.where(mask, c2[e] + pc - 1, r2)
                c2[e] = c2[e] + jnp.sum(mi)
            r1l[dsl] = r1
            r2l[dsl] = r2

        cv = jnp.zeros((L,), jnp.int32)
        for e in range(E):
            cv = jnp.where(iota == e, c1[e], cv)
            cv = jnp.where(iota == (E + e), c2[e], cv)
        cnts_v[...] = cv
        pltpu.sync_copy(cnts_v, cnts_sh.at[sid])
        plsc.subcore_barrier()

        pltpu.sync_copy(cnts_sh, allc_v)
        sv = jnp.broadcast_to(sid, (L,)).astype(jnp.int32)
        pref = jnp.zeros((L,), jnp.int32)
        tot = jnp.zeros((L,), jnp.int32)
        for t in range(NS):
            row = allc_v[t]
            tv = jnp.full((L,), t, jnp.int32)
            pref = pref + jnp.where(tv < sv, row, 0)
            tot = tot + row
        tot_v[...] = tot
        sh = plsc.load_gather(tot_v, [jnp.maximum(iota - E, 0)])
        off = pref + jnp.where(iota >= E, sh, 0)
        off_v[...] = off

        for c in range(n_loc):
            dsl = pl.ds(c * L, L)
            idx1 = e1l[dsl]
            idx2 = e2l[dsl]
            base1 = plsc.load_gather(off_v, [idx1])
            base2 = plsc.load_gather(off_v, [idx2 + E])
            l1 = base1 + r1l[dsl]
            l2 = base2 + r2l[dsl]
            k1 = l1 < C
            k2 = l2 < C
            g1k = jnp.where(k1, g1l[dsl], 0.0)
            g2k = jnp.where(k2, g2l[dsl], 0.0)
            rd = 1.0 / jnp.maximum(g1k + g2k, F32_EPS)
            d1 = idx1 * C + l1
            d2 = idx2 * C + l2
            d1m_v[dsl] = jnp.where(k1, d1, slots)
            d2m_v[dsl] = jnp.where(k2, d2, slots)
            d1o_v[dsl] = jnp.where(k1, d1, 0)
            d2o_v[dsl] = jnp.where(k2, d2, 0)
            w1o_v[dsl] = g1k * rd
            w2o_v[dsl] = g2k * rd

        @pl.when(cid == 0)
        def _():
            pltpu.sync_copy(d1o_v, dst1_hbm.at[pl.ds(s0, tpt)])
            pltpu.sync_copy(d2o_v, dst2_hbm.at[pl.ds(s0, tpt)])
            pltpu.sync_copy(w1o_v, w1_hbm.at[pl.ds(s0, tpt)])
            pltpu.sync_copy(w2o_v, w2_hbm.at[pl.ds(s0, tpt)])

        pltpu.sync_copy(d1m_v, d1_sh.at[pl.ds(s0, tpt)])
        pltpu.sync_copy(d2m_v, d2_sh.at[pl.ds(s0, tpt)])
        plsc.subcore_barrier()

        pltpu.sync_copy(d1_sh, d1a_v)
        pltpu.sync_copy(d2_sh, d2a_v)

        def zinit(i, carry):
            src_v[pl.ds(pl.multiple_of(i * L, L), L)] = jnp.zeros(
                (L,), jnp.int32)
            return carry

        lax.fori_loop(0, slots // L, zinit, 0)

        def rebuild(i, carry):
            ds = pl.ds(pl.multiple_of(i * L, L), L)
            tid = (i * L + iota).astype(jnp.int32)
            d1c = d1a_v[ds]
            plsc.store_scatter(src_v, [d1c], tid, mask=d1c < slots)
            d2c = d2a_v[ds]
            plsc.store_scatter(src_v, [d2c], tid, mask=d2c < slots)
            return carry

        lax.fori_loop(0, S // L, rebuild, 0)

        base = wid * spw
        nchunks = spw // rows_chunk
        bufs = [rows_v, rows2_v]
        rsems = [semr0, semr1]
        wsems = [semw0, semw1]

        def rd(j):
            return pltpu.async_copy(
                x_ref.at[src_v.at[pl.ds(base + j * rows_chunk, rows_chunk)]],
                bufs[j % 2], rsems[j % 2])

        def wr(j):
            return pltpu.async_copy(
                bufs[j % 2],
                disp_ref.at[pl.ds(base + j * rows_chunk, rows_chunk)],
                wsems[j % 2])

        rds = [rd(0), rd(1)]
        wrs = [None, None]
        for j in range(nchunks):
            rds[j % 2].wait()
            wrs[j % 2] = wr(j)
            if j + 2 < nchunks:
                wrs[j % 2].wait()
                rds[j % 2] = rd(j + 2)
        wrs[(nchunks - 2) % 2].wait()
        wrs[(nchunks - 1) % 2].wait()

    mesh = plsc.VectorSubcoreMesh(core_axis_name="c", subcore_axis_name="s")
    return pl.kernel(
        body,
        out_type=[
            jax.ShapeDtypeStruct((slots, M), jnp.float32),
            jax.ShapeDtypeStruct((S,), jnp.int32),
            jax.ShapeDtypeStruct((S,), jnp.int32),
            jax.ShapeDtypeStruct((S,), jnp.float32),
            jax.ShapeDtypeStruct((S,), jnp.float32),
        ],
        mesh=mesh,
        compiler_params=pltpu.CompilerParams(needs_layout_passes=False),
        scratch_types=[
            pltpu.VMEM((E, S), jnp.float32),
            pltpu.VMEM((E, S), jnp.float32),
            pltpu.VMEM((tpt,), jnp.int32),
            pltpu.VMEM((tpt,), jnp.int32),
            pltpu.VMEM((tpt,), jnp.float32),
            pltpu.VMEM((tpt,), jnp.float32),
            pltpu.VMEM((tpt,), jnp.int32),
            pltpu.VMEM((tpt,), jnp.int32),
            pltpu.VMEM((L,), jnp.int32),
            pltpu.VMEM((NS, L), jnp.int32),
            pltpu.VMEM((L,), jnp.int32),
            pltpu.VMEM((L,), jnp.int32),
            pltpu.VMEM((tpt,), jnp.int32),
            pltpu.VMEM((tpt,), jnp.int32),
            pltpu.VMEM((tpt,), jnp.int32),
            pltpu.VMEM((tpt,), jnp.int32),
            pltpu.VMEM((tpt,), jnp.float32),
            pltpu.VMEM((tpt,), jnp.float32),
            pltpu.VMEM((S,), jnp.int32),
            pltpu.VMEM((S,), jnp.int32),
            pltpu.VMEM((slots,), jnp.int32),
            pltpu.VMEM((rows_chunk, M), jnp.float32),
            pltpu.VMEM((rows_chunk, M), jnp.float32),
            pltpu.VMEM_SHARED((NS, L), jnp.int32),
            pltpu.VMEM_SHARED((S,), jnp.int32),
            pltpu.VMEM_SHARED((S,), jnp.int32),
            pltpu.SemaphoreType.DMA,
            pltpu.SemaphoreType.DMA,
            pltpu.SemaphoreType.DMA,
            pltpu.SemaphoreType.DMA,
        ],
    )


def _expert_body(disp_ref, We_ref, be_ref, eo_ref):
    part = lax.dot_general(disp_ref[...].astype(jnp.bfloat16),
                           We_ref[0].astype(jnp.bfloat16),
                           (((1,), (1,)), ((), ())),
                           preferred_element_type=jnp.float32)
    k = pl.program_id(1)

    @pl.when(k == 0)
    def _():
        eo_ref[...] = part + be_ref[0]

    @pl.when(k > 0)
    def _():
        eo_ref[...] = eo_ref[...] + part


def _make_combine(S, C, E, Dout):
    per_w = S // NW
    tchunk = 16
    nch = per_w // tchunk

    def body(eo_ref, dst1_ref, dst2_ref, w1_ref, w2_ref, y_ref,
             d1_v, d2_v, w1_v, w2_v,
             r1a, r1b, r2a, r2b, outa, outb,
             sg1a, sg1b, sg2a, sg2b, swa, swb):
        wid = lax.axis_index("s") * NC + lax.axis_index("c")
        base = wid * per_w
        pltpu.sync_copy(dst1_ref.at[pl.ds(base, per_w)], d1_v)
        pltpu.sync_copy(dst2_ref.at[pl.ds(base, per_w)], d2_v)
        pltpu.sync_copy(w1_ref.at[pl.ds(base, per_w)], w1_v)
        pltpu.sync_copy(w2_ref.at[pl.ds(base, per_w)], w2_v)
        r1s = [r1a, r1b]
        r2s = [r2a, r2b]
        outs = [outa, outb]
        sg1 = [sg1a, sg1b]
        sg2 = [sg2a, sg2b]
        sw = [swa, swb]

        def gath(h):
            b = h % 2
            c1 = pltpu.async_copy(
                eo_ref.at[d1_v.at[pl.ds(h * tchunk, tchunk)]], r1s[b], sg1[b])
            c2 = pltpu.async_copy(
                eo_ref.at[d2_v.at[pl.ds(h * tchunk, tchunk)]], r2s[b], sg2[b])
            return (c1, c2)

        gs = [gath(0), gath(1)]
        ws = [None, None]
        for h in range(nch):
            b = h % 2
            gs[b][0].wait()
            gs[b][1].wait()
            if h >= 2:
                ws[b].wait()

            def tok(t, carry):
                widx = jnp.broadcast_to(h * tchunk + t, (L,)).astype(jnp.int32)
                w1b = plsc.load_gather(w1_v, [widx])
                w2b = plsc.load_gather(w2_v, [widx])
                for k in range(Dout // L):
                    dsk = pl.ds(k * L, L)
                    outs[b][t, dsk] = (r1s[b][t, dsk] * w1b +
                                       r2s[b][t, dsk] * w2b)
                return carry

            lax.fori_loop(0, tchunk, tok, 0)
            ws[b] = pltpu.async_copy(
                outs[b], y_ref.at[pl.ds(base + h * tchunk, tchunk)], sw[b])
            if h + 2 < nch:
                gs[b] = gath(h + 2)
        ws[(nch - 2) % 2].wait()
        ws[(nch - 1) % 2].wait()

    mesh = plsc.VectorSubcoreMesh(core_axis_name="c", subcore_axis_name="s")
    return pl.kernel(
        body,
        out_type=jax.ShapeDtypeStruct((S, Dout), jnp.float32),
        mesh=mesh,
        compiler_params=pltpu.CompilerParams(needs_layout_passes=False),
        scratch_types=[
            pltpu.VMEM((per_w,), jnp.int32),
            pltpu.VMEM((per_w,), jnp.int32),
            pltpu.VMEM((per_w,), jnp.float32),
            pltpu.VMEM((per_w,), jnp.float32),
            pltpu.VMEM((tchunk, Dout), jnp.float32),
            pltpu.VMEM((tchunk, Dout), jnp.float32),
            pltpu.VMEM((tchunk, Dout), jnp.float32),
            pltpu.VMEM((tchunk, Dout), jnp.float32),
            pltpu.VMEM((tchunk, Dout), jnp.float32),
            pltpu.VMEM((tchunk, Dout), jnp.float32),
            pltpu.SemaphoreType.DMA,
            pltpu.SemaphoreType.DMA,
            pltpu.SemaphoreType.DMA,
            pltpu.SemaphoreType.DMA,
            pltpu.SemaphoreType.DMA,
            pltpu.SemaphoreType.DMA,
        ],
    )


@jax.jit
def kernel(x, wg, We, be, gnoise):
    seq, tok, M = x.shape
    S = seq * tok
    E = wg.shape[0]
    Dout = We.shape[1]
    C = 2 * S // E

    xf = x.reshape(S, M)
    gnT = gnoise.T

    logT, noisyT = pl.pallas_call(
        _logits_body,
        out_shape=[jax.ShapeDtypeStruct((E, S), jnp.float32),
                   jax.ShapeDtypeStruct((E, S), jnp.float32)],
    )(xf, wg, gnT)

    disp, dst1, dst2, w1, w2 = _make_gate_dispatch(E, S, C, M)(
        logT, noisyT, xf)

    KB = 4
    eo = pl.pallas_call(
        _expert_body,
        grid=(E, KB),
        in_specs=[
            pl.BlockSpec((C, M // KB), lambda e, k: (e, k)),
            pl.BlockSpec((1, Dout, M // KB), lambda e, k: (e, 0, k)),
            pl.BlockSpec((1, 1, Dout), lambda e, k: (e, 0, 0)),
        ],
        out_specs=pl.BlockSpec((C, Dout), lambda e, k: (e, 0)),
        out_shape=jax.ShapeDtypeStruct((E * C, Dout), jnp.float32),
    )(disp, We, be.reshape(E, 1, Dout))

    y = _make_combine(S, C, E, Dout)(eo, dst1, dst2, w1, w2)
    return y.reshape(x.shape)


# speedup vs baseline: 1.1970x; 1.1077x over previous
import jax
import jax.numpy as jnp
from jax import lax
from jax.experimental import pallas as pl
from jax.experimental.pallas import tpu as pltpu
from jax.experimental.pallas import tpu_sc as plsc

NC, NS, L = 2, 16, 16
NW = NC * NS
F32_EPS = 1.1920929e-07


def _logits_body(x_ref, wg_ref, gnT_ref, logT_ref, noisyT_ref):
    lg = lax.dot_general(wg_ref[...], x_ref[...], (((1,), (1,)), ((), ())),
                         preferred_element_type=jnp.float32)
    logT_ref[...] = lg
    noisyT_ref[...] = lg + gnT_ref[...]


def _make_gate(E, S, C):
    slots = E * C
    tpt = S // NS
    n_loc = tpt // L

    def body(logT, noisyT,
             src_hbm, wsl_hbm,
             lg_v, ns_v, e1l, e2l, g1l, g2l, r1l, r2l,
             cnts_v, allc_v, tot_v, off_v,
             d1m_v, d2m_v, d1o_v, d2o_v, w1o_v, w2o_v,
             d1a_v, d2a_v, w1a_v, w2a_v, src_v, wsl_v,
             cnts_sh, d1_sh, d2_sh, w1_sh, w2_sh):
        cid = lax.axis_index("c")
        sid = lax.axis_index("s")
        wid = sid * NC + cid
        s0 = sid * tpt
        iota = lax.iota(jnp.int32, L)

        pltpu.sync_copy(logT, lg_v)
        pltpu.sync_copy(noisyT, ns_v)

        c1 = [jnp.int32(0)] * E
        c2 = [jnp.int32(0)] * E
        for c in range(n_loc):
            ds = pl.ds(s0 + c * L, L)
            dsl = pl.ds(c * L, L)
            ls = [lg_v[e, ds] for e in range(E)]
            m = ls[0]
            for e in range(1, E):
                m = jnp.maximum(m, ls[e])
            exs = [jnp.exp(ls[e] - m) for e in range(E)]
            Z = exs[0]
            for e in range(1, E):
                Z = Z + exs[e]
            idx1 = jnp.full((L,), 127, jnp.int32)
            for e in range(E):
                idx1 = jnp.minimum(idx1, jnp.where(ls[e] == m, e, 127))
            g1 = jnp.zeros((L,), jnp.float32)
            for e in range(E):
                g1 = jnp.where(idx1 == e, exs[e], g1)
            neg = jnp.full((L,), -jnp.inf, jnp.float32)
            nss = []
            m2 = neg
            for e in range(E):
                nv = jnp.where(idx1 == e, neg, ns_v[e, ds])
                nss.append(nv)
                m2 = jnp.maximum(m2, nv)
            idx2 = jnp.full((L,), 127, jnp.int32)
            for e in range(E):
                idx2 = jnp.minimum(idx2, jnp.where(nss[e] == m2, e, 127))
            g2 = jnp.zeros((L,), jnp.float32)
            for e in range(E):
                g2 = jnp.where(idx2 == e, exs[e], g2)
            rZ = 1.0 / Z
            e1l[dsl] = idx1
            e2l[dsl] = idx2
            g1l[dsl] = g1 * rZ
            g2l[dsl] = g2 * rZ
            r1 = jnp.zeros((L,), jnp.int32)
            r2 = jnp.zeros((L,), jnp.int32)
            for e in range(E):
                mask = idx1 == e
                mi = jnp.where(mask, 1, 0).astype(jnp.int32)
                pc = plsc.cumsum(mi)
                r1 = jnp.where(mask, c1[e] + pc - 1, r1)
                c1[e] = c1[e] + jnp.sum(mi)
                mask = idx2 == e
                mi = jnp.where(mask, 1, 0).astype(jnp.int32)
                pc = plsc.cumsum(mi)
                r2 = jnp.where(mask, c2[e] + pc - 1, r2)
                c2[e] = c2[e] + jnp.sum(mi)
            r1l[dsl] = r1
            r2l[dsl] = r2

        cv = jnp.zeros((L,), jnp.int32)
        for e in range(E):
            cv = jnp.where(iota == e, c1[e], cv)
            cv = jnp.where(iota == (E + e), c2[e], cv)
        cnts_v[...] = cv
        pltpu.sync_copy(cnts_v, cnts_sh.at[sid])
        plsc.subcore_barrier()

        pltpu.sync_copy(cnts_sh, allc_v)
        sv = jnp.broadcast_to(sid, (L,)).astype(jnp.int32)
        pref = jnp.zeros((L,), jnp.int32)
        tot = jnp.zeros((L,), jnp.int32)
        for t in range(NS):
            row = allc_v[t]
            tv = jnp.full((L,), t, jnp.int32)
            pref = pref + jnp.where(tv < sv, row, 0)
            tot = tot + row
        tot_v[...] = tot
        sh = plsc.load_gather(tot_v, [jnp.maximum(iota - E, 0)])
        off = pref + jnp.where(iota >= E, sh, 0)
        off_v[...] = off

        for c in range(n_loc):
            dsl = pl.ds(c * L, L)
            idx1 = e1l[dsl]
            idx2 = e2l[dsl]
            base1 = plsc.load_gather(off_v, [idx1])
            base2 = plsc.load_gather(off_v, [idx2 + E])
            l1 = base1 + r1l[dsl]
            l2 = base2 + r2l[dsl]
            k1 = l1 < C
            k2 = l2 < C
            g1k = jnp.where(k1, g1l[dsl], 0.0)
            g2k = jnp.where(k2, g2l[dsl], 0.0)
            rd = 1.0 / jnp.maximum(g1k + g2k, F32_EPS)
            d1 = idx1 * C + l1
            d2 = idx2 * C + l2
            d1m_v[dsl] = jnp.where(k1, d1, slots)
            d2m_v[dsl] = jnp.where(k2, d2, slots)
            d1o_v[dsl] = jnp.where(k1, d1, 0)
            d2o_v[dsl] = jnp.where(k2, d2, 0)
            w1o_v[dsl] = g1k * rd
            w2o_v[dsl] = g2k * rd

        pltpu.sync_copy(d1m_v, d1_sh.at[pl.ds(s0, tpt)])
        pltpu.sync_copy(d2m_v, d2_sh.at[pl.ds(s0, tpt)])
        pltpu.sync_copy(w1o_v, w1_sh.at[pl.ds(s0, tpt)])
        pltpu.sync_copy(w2o_v, w2_sh.at[pl.ds(s0, tpt)])
        plsc.subcore_barrier()

        pltpu.sync_copy(d1_sh, d1a_v)
        pltpu.sync_copy(d2_sh, d2a_v)
        pltpu.sync_copy(w1_sh, w1a_v)
        pltpu.sync_copy(w2_sh, w2a_v)

        def zinit(i, carry):
            ds = pl.ds(pl.multiple_of(i * L, L), L)
            src_v[ds] = jnp.zeros((L,), jnp.int32)
            wsl_v[ds] = jnp.zeros((L,), jnp.float32)
            return carry

        lax.fori_loop(0, slots // L, zinit, 0)

        def rebuild(i, carry):
            ds = pl.ds(pl.multiple_of(i * L, L), L)
            tid = (i * L + iota).astype(jnp.int32)
            d1c = d1a_v[ds]
            plsc.store_scatter(src_v, [d1c], tid, mask=d1c < slots)
            plsc.store_scatter(wsl_v, [d1c], w1a_v[ds], mask=d1c < slots)
            d2c = d2a_v[ds]
            plsc.store_scatter(src_v, [d2c], tid, mask=d2c < slots)
            plsc.store_scatter(wsl_v, [d2c], w2a_v[ds], mask=d2c < slots)
            return carry

        lax.fori_loop(0, S // L, rebuild, 0)

        spt = slots // NS
        @pl.when(cid == 0)
        def _():
            pltpu.sync_copy(src_v.at[pl.ds(sid * spt, spt)],
                            src_hbm.at[pl.ds(sid * spt, spt)])
            pltpu.sync_copy(wsl_v.at[pl.ds(sid * spt, spt)],
                            wsl_hbm.at[pl.ds(sid * spt, spt)])

    mesh = plsc.VectorSubcoreMesh(core_axis_name="c", subcore_axis_name="s")
    return pl.kernel(
        body,
        out_type=[
            jax.ShapeDtypeStruct((slots,), jnp.int32),
            jax.ShapeDtypeStruct((slots,), jnp.float32),
        ],
        mesh=mesh,
        compiler_params=pltpu.CompilerParams(needs_layout_passes=False),
        scratch_types=[
            pltpu.VMEM((E, S), jnp.float32),
            pltpu.VMEM((E, S), jnp.float32),
            pltpu.VMEM((tpt,), jnp.int32),
            pltpu.VMEM((tpt,), jnp.int32),
            pltpu.VMEM((tpt,), jnp.float32),
            pltpu.VMEM((tpt,), jnp.float32),
            pltpu.VMEM((tpt,), jnp.int32),
            pltpu.VMEM((tpt,), jnp.int32),
            pltpu.VMEM((L,), jnp.int32),
            pltpu.VMEM((NS, L), jnp.int32),
            pltpu.VMEM((L,), jnp.int32),
            pltpu.VMEM((L,), jnp.int32),
            pltpu.VMEM((tpt,), jnp.int32),
            pltpu.VMEM((tpt,), jnp.int32),
            pltpu.VMEM((tpt,), jnp.int32),
            pltpu.VMEM((tpt,), jnp.int32),
            pltpu.VMEM((tpt,), jnp.float32),
            pltpu.VMEM((tpt,), jnp.float32),
            pltpu.VMEM((S,), jnp.int32),
            pltpu.VMEM((S,), jnp.int32),
            pltpu.VMEM((S,), jnp.float32),
            pltpu.VMEM((S,), jnp.float32),
            pltpu.VMEM((slots,), jnp.int32),
            pltpu.VMEM((slots,), jnp.float32),
            pltpu.VMEM_SHARED((NS, L), jnp.int32),
            pltpu.VMEM_SHARED((S,), jnp.int32),
            pltpu.VMEM_SHARED((S,), jnp.int32),
            pltpu.VMEM_SHARED((S,), jnp.float32),
            pltpu.VMEM_SHARED((S,), jnp.float32),
        ],
    )


def _make_mega(E, S, C, M, Dout):
    def body(x_ref, src_ref, wsl_ref, We_ref, be_ref, y_ref, xbf_ref):
        e = pl.program_id(0)

        @pl.when(e == 0)
        def _():
            xbf_ref[...] = x_ref[...].astype(jnp.bfloat16)

        srccol = src_ref[0]
        iota2 = lax.broadcasted_iota(jnp.int32, (C, S), 1)
        P = jnp.where(iota2 == srccol, 1.0, 0.0).astype(jnp.bfloat16)
        dispb = lax.dot_general(
            P, xbf_ref[...], (((1,), (0,)), ((), ())),
            preferred_element_type=jnp.float32).astype(jnp.bfloat16)
        acc = lax.dot_general(dispb, We_ref[0].astype(jnp.bfloat16),
                              (((1,), (1,)), ((), ())),
                              preferred_element_type=jnp.float32)
        scaled = (wsl_ref[0] * (acc + be_ref[0])).astype(jnp.bfloat16)
        part = lax.dot_general(P, scaled, (((0,), (0,)), ((), ())),
                               preferred_element_type=jnp.float32)

        @pl.when(e == 0)
        def _():
            y_ref[...] = part

        @pl.when(e > 0)
        def _():
            y_ref[...] = y_ref[...] + part

    return pl.pallas_call(
        body,
        grid=(E,),
        in_specs=[
            pl.BlockSpec((S, M), lambda e: (0, 0)),
            pl.BlockSpec((1, C, 1), lambda e: (e, 0, 0)),
            pl.BlockSpec((1, C, 1), lambda e: (e, 0, 0)),
            pl.BlockSpec((1, Dout, M), lambda e: (e, 0, 0)),
            pl.BlockSpec((1, 1, Dout), lambda e: (e, 0, 0)),
        ],
        out_specs=pl.BlockSpec((S, Dout), lambda e: (0, 0)),
        out_shape=jax.ShapeDtypeStruct((S, Dout), jnp.float32),
        scratch_shapes=[pltpu.VMEM((S, M), jnp.bfloat16)],
    )


def _make_combine(S, C, E, Dout):
    per_w = S // NW
    tchunk = 16
    nch = per_w // tchunk

    def body(eo_ref, dst1_ref, dst2_ref, w1_ref, w2_ref, y_ref,
             d1_v, d2_v, w1_v, w2_v,
             r1a, r1b, r2a, r2b, outa, outb,
             sg1a, sg1b, sg2a, sg2b, swa, swb):
        wid = lax.axis_index("s") * NC + lax.axis_index("c")
        base = wid * per_w
        pltpu.sync_copy(dst1_ref.at[pl.ds(base, per_w)], d1_v)
        pltpu.sync_copy(dst2_ref.at[pl.ds(base, per_w)], d2_v)
        pltpu.sync_copy(w1_ref.at[pl.ds(base, per_w)], w1_v)
        pltpu.sync_copy(w2_ref.at[pl.ds(base, per_w)], w2_v)
        r1s = [r1a, r1b]
        r2s = [r2a, r2b]
        outs = [outa, outb]
        sg1 = [sg1a, sg1b]
        sg2 = [sg2a, sg2b]
        sw = [swa, swb]

        def gath(h):
            b = h % 2
            c1 = pltpu.async_copy(
                eo_ref.at[d1_v.at[pl.ds(h * tchunk, tchunk)]], r1s[b], sg1[b])
            c2 = pltpu.async_copy(
                eo_ref.at[d2_v.at[pl.ds(h * tchunk, tchunk)]], r2s[b], sg2[b])
            return (c1, c2)

        gs = [gath(0), gath(1)]
        ws = [None, None]
        for h in range(nch):
            b = h % 2
            gs[b][0].wait()
            gs[b][1].wait()
            if h >= 2:
                ws[b].wait()

            def tok(t, carry):
                widx = jnp.broadcast_to(h * tchunk + t, (L,)).astype(jnp.int32)
                w1b = plsc.load_gather(w1_v, [widx])
                w2b = plsc.load_gather(w2_v, [widx])
                for k in range(Dout // L):
                    dsk = pl.ds(k * L, L)
                    outs[b][t, dsk] = (r1s[b][t, dsk] * w1b +
                                       r2s[b][t, dsk] * w2b)
                return carry

            lax.fori_loop(0, tchunk, tok, 0)
            ws[b] = pltpu.async_copy(
                outs[b], y_ref.at[pl.ds(base + h * tchunk, tchunk)], sw[b])
            if h + 2 < nch:
                gs[b] = gath(h + 2)
        ws[(nch - 2) % 2].wait()
        ws[(nch - 1) % 2].wait()

    mesh = plsc.VectorSubcoreMesh(core_axis_name="c", subcore_axis_name="s")
    return pl.kernel(
        body,
        out_type=jax.ShapeDtypeStruct((S, Dout), jnp.float32),
        mesh=mesh,
        compiler_params=pltpu.CompilerParams(needs_layout_passes=False),
        scratch_types=[
            pltpu.VMEM((per_w,), jnp.int32),
            pltpu.VMEM((per_w,), jnp.int32),
            pltpu.VMEM((per_w,), jnp.float32),
            pltpu.VMEM((per_w,), jnp.float32),
            pltpu.VMEM((tchunk, Dout), jnp.float32),
            pltpu.VMEM((tchunk, Dout), jnp.float32),
            pltpu.VMEM((tchunk, Dout), jnp.float32),
            pltpu.VMEM((tchunk, Dout), jnp.float32),
            pltpu.VMEM((tchunk, Dout), jnp.float32),
            pltpu.VMEM((tchunk, Dout), jnp.float32),
            pltpu.SemaphoreType.DMA,
            pltpu.SemaphoreType.DMA,
            pltpu.SemaphoreType.DMA,
            pltpu.SemaphoreType.DMA,
            pltpu.SemaphoreType.DMA,
            pltpu.SemaphoreType.DMA,
        ],
    )


@jax.jit
def kernel(x, wg, We, be, gnoise):
    seq, tok, M = x.shape
    S = seq * tok
    E = wg.shape[0]
    Dout = We.shape[1]
    C = 2 * S // E

    xf = x.reshape(S, M)
    gnT = gnoise.T

    logT, noisyT = pl.pallas_call(
        _logits_body,
        out_shape=[jax.ShapeDtypeStruct((E, S), jnp.float32),
                   jax.ShapeDtypeStruct((E, S), jnp.float32)],
    )(xf, wg, gnT)

    src, wsl = _make_gate(E, S, C)(logT, noisyT)

    y = _make_mega(E, S, C, M, Dout)(
        xf, src.reshape(E, C, 1), wsl.reshape(E, C, 1),
        We, be.reshape(E, 1, Dout))
    return y.reshape(x.shape)


# speedup vs baseline: 1.3691x; 1.1438x over previous
import jax
import jax.numpy as jnp
from jax import lax
from jax.experimental import pallas as pl
from jax.experimental.pallas import tpu as pltpu
from jax.experimental.pallas import tpu_sc as plsc

NC, NS, L = 2, 16, 16
NW = NC * NS
F32_EPS = 1.1920929e-07


def _logits_body(x_ref, wg_ref, gnT_ref, logT_ref, noisyT_ref):
    lg = lax.dot_general(wg_ref[...], x_ref[...], (((1,), (1,)), ((), ())),
                         preferred_element_type=jnp.float32)
    logT_ref[...] = lg
    noisyT_ref[...] = lg + gnT_ref[...]


def _make_gate_dispatch(E, S, C, M):
    slots = E * C
    tpt = S // NS
    spw = slots // NW
    n_loc = tpt // L
    rows_chunk = 32

    def body(logT, noisyT, x_ref,
             disp_ref, dst1_hbm, dst2_hbm, w1_hbm, w2_hbm,
             lg_v, ns_v, e1l, e2l, g1l, g2l, r1l, r2l,
             cnts_v, allc_v, tot_v, off_v,
             d1m_v, d2m_v, d1o_v, d2o_v, w1o_v, w2o_v,
             d1a_v, d2a_v, src_v, rows_v, rows2_v,
             cnts_sh, d1_sh, d2_sh, semr0, semr1, semw0, semw1):
        cid = lax.axis_index("c")
        sid = lax.axis_index("s")
        wid = sid * NC + cid
        s0 = sid * tpt
        iota = lax.iota(jnp.int32, L)

        pltpu.sync_copy(logT.at[:, pl.ds(s0, tpt)], lg_v)
        pltpu.sync_copy(noisyT.at[:, pl.ds(s0, tpt)], ns_v)

        c1 = [jnp.int32(0)] * E
        c2 = [jnp.int32(0)] * E
        for c in range(n_loc):
            ds = pl.ds(c * L, L)
            dsl = ds
            ls = [lg_v[e, ds] for e in range(E)]
            m = ls[0]
            for e in range(1, E):
                m = jnp.maximum(m, ls[e])
            exs = [jnp.exp(ls[e] - m) for e in range(E)]
            Z = exs[0]
            for e in range(1, E):
                Z = Z + exs[e]
            idx1 = jnp.full((L,), 127, jnp.int32)
            for e in range(E):
                idx1 = jnp.minimum(idx1, jnp.where(ls[e] == m, e, 127))
            g1 = jnp.zeros((L,), jnp.float32)
            for e in range(E):
                g1 = jnp.where(idx1 == e, exs[e], g1)
            neg = jnp.full((L,), -jnp.inf, jnp.float32)
            nss = []
            m2 = neg
            for e in range(E):
                nv = jnp.where(idx1 == e, neg, ns_v[e, ds])
                nss.append(nv)
                m2 = jnp.maximum(m2, nv)
            idx2 = jnp.full((L,), 127, jnp.int32)
            for e in range(E):
                idx2 = jnp.minimum(idx2, jnp.where(nss[e] == m2, e, 127))
            g2 = jnp.zeros((L,), jnp.float32)
            for e in range(E):
                g2 = jnp.where(idx2 == e, exs[e], g2)
            rZ = 1.0 / Z
            e1l[dsl] = idx1
            e2l[dsl] = idx2
            g1l[dsl] = g1 * rZ
            g2l[dsl] = g2 * rZ
            r1 = jnp.zeros((L,), jnp.int32)
            r2 = jnp.zeros((L,), jnp.int32)
            for e in range(E):
                mask = idx1 == e
                mi = jnp.where(mask, 1, 0).astype(jnp.int32)
                pc = plsc.cumsum(mi)
                r1 = jnp.where(mask, c1[e] + pc - 1, r1)
                c1[e] = c1[e] + jnp.sum(mi)
                mask = idx2 == e
                mi = jnp.where(mask, 1, 0).astype(jnp.int32)
                pc = plsc.cumsum(mi)
                r2 = jnp.where(mask, c2[e] + pc - 1, r2)
                c2[e] = c2[e] + jnp.sum(mi)
            r1l[dsl] = r1
            r2l[dsl] = r2

        cv = jnp.zeros((L,), jnp.int32)
        for e in range(E):
            cv = jnp.where(iota == e, c1[e], cv)
            cv = jnp.where(iota == (E + e), c2[e], cv)
        cnts_v[...] = cv
        pltpu.sync_copy(cnts_v, cnts_sh.at[sid])
        plsc.subcore_barrier()

        pltpu.sync_copy(cnts_sh, allc_v)
        sv = jnp.broadcast_to(sid, (L,)).astype(jnp.int32)
        pref = jnp.zeros((L,), jnp.int32)
        tot = jnp.zeros((L,), jnp.int32)
        for t in range(NS):
            row = allc_v[t]
            tv = jnp.full((L,), t, jnp.int32)
            pref = pref + jnp.where(tv < sv, row, 0)
            tot = tot + row
        tot_v[...] = tot
        sh = plsc.load_gather(tot_v, [jnp.maximum(iota - E, 0)])
        off = pref + jnp.where(iota >= E, sh, 0)
        off_v[...] = off

        for c in range(n_loc):
            dsl = pl.ds(c * L, L)
            idx1 = e1l[dsl]
            idx2 = e2l[dsl]
            base1 = plsc.load_gather(off_v, [idx1])
            base2 = plsc.load_gather(off_v, [idx2 + E])
            l1 = base1 + r1l[dsl]
            l2 = base2 + r2l[dsl]
            k1 = l1 < C
            k2 = l2 < C
            g1k = jnp.where(k1, g1l[dsl], 0.0)
            g2k = jnp.where(k2, g2l[dsl], 0.0)
            rd = 1.0 / jnp.maximum(g1k + g2k, F32_EPS)
            d1 = idx1 * C + l1
            d2 = idx2 * C + l2
            d1m_v[dsl] = jnp.where(k1, d1, slots)
            d2m_v[dsl] = jnp.where(k2, d2, slots)
            d1o_v[dsl] = jnp.where(k1, d1, 0)
            d2o_v[dsl] = jnp.where(k2, d2, 0)
            w1o_v[dsl] = g1k * rd
            w2o_v[dsl] = g2k * rd

        @pl.when(cid == 0)
        def _():
            pltpu.sync_copy(d1o_v, dst1_hbm.at[pl.ds(s0, tpt)])
            pltpu.sync_copy(d2o_v, dst2_hbm.at[pl.ds(s0, tpt)])
            pltpu.sync_copy(w1o_v, w1_hbm.at[pl.ds(s0, tpt)])
            pltpu.sync_copy(w2o_v, w2_hbm.at[pl.ds(s0, tpt)])

        pltpu.sync_copy(d1m_v, d1_sh.at[pl.ds(s0, tpt)])
        pltpu.sync_copy(d2m_v, d2_sh.at[pl.ds(s0, tpt)])
        plsc.subcore_barrier()

        pltpu.sync_copy(d1_sh, d1a_v)
        pltpu.sync_copy(d2_sh, d2a_v)

        def zinit(i, carry):
            src_v[pl.ds(pl.multiple_of(i * L, L), L)] = jnp.zeros(
                (L,), jnp.int32)
            return carry

        lax.fori_loop(0, slots // L, zinit, 0)

        def rebuild(i, carry):
            ds = pl.ds(pl.multiple_of(i * L, L), L)
            tid = (i * L + iota).astype(jnp.int32)
            d1c = d1a_v[ds]
            plsc.store_scatter(src_v, [d1c], tid, mask=d1c < slots)
            d2c = d2a_v[ds]
            plsc.store_scatter(src_v, [d2c], tid, mask=d2c < slots)
            return carry

        lax.fori_loop(0, S // L, rebuild, 0)

        base = wid * spw
        nchunks = spw // rows_chunk
        bufs = [rows_v, rows2_v]
        rsems = [semr0, semr1]
        wsems = [semw0, semw1]

        def rd(j):
            return pltpu.async_copy(
                x_ref.at[src_v.at[pl.ds(base + j * rows_chunk, rows_chunk)]],
                bufs[j % 2], rsems[j % 2])

        def wr(j):
            return pltpu.async_copy(
                bufs[j % 2],
                disp_ref.at[pl.ds(base + j * rows_chunk, rows_chunk)],
                wsems[j % 2])

        rds = [rd(0), rd(1)]
        wrs = [None, None]
        for j in range(nchunks):
            rds[j % 2].wait()
            wrs[j % 2] = wr(j)
            if j + 2 < nchunks:
                wrs[j % 2].wait()
                rds[j % 2] = rd(j + 2)
        wrs[(nchunks - 2) % 2].wait()
        wrs[(nchunks - 1) % 2].wait()

    mesh = plsc.VectorSubcoreMesh(core_axis_name="c", subcore_axis_name="s")
    return pl.kernel(
        body,
        out_type=[
            jax.ShapeDtypeStruct((slots, M), jnp.float32),
            jax.ShapeDtypeStruct((S,), jnp.int32),
            jax.ShapeDtypeStruct((S,), jnp.int32),
            jax.ShapeDtypeStruct((S,), jnp.float32),
            jax.ShapeDtypeStruct((S,), jnp.float32),
        ],
        mesh=mesh,
        compiler_params=pltpu.CompilerParams(needs_layout_passes=False),
        scratch_types=[
            pltpu.VMEM((E, tpt), jnp.float32),
            pltpu.VMEM((E, tpt), jnp.float32),
            pltpu.VMEM((tpt,), jnp.int32),
            pltpu.VMEM((tpt,), jnp.int32),
            pltpu.VMEM((tpt,), jnp.float32),
            pltpu.VMEM((tpt,), jnp.float32),
            pltpu.VMEM((tpt,), jnp.int32),
            pltpu.VMEM((tpt,), jnp.int32),
            pltpu.VMEM((L,), jnp.int32),
            pltpu.VMEM((NS, L), jnp.int32),
            pltpu.VMEM((L,), jnp.int32),
            pltpu.VMEM((L,), jnp.int32),
            pltpu.VMEM((tpt,), jnp.int32),
            pltpu.VMEM((tpt,), jnp.int32),
            pltpu.VMEM((tpt,), jnp.int32),
            pltpu.VMEM((tpt,), jnp.int32),
            pltpu.VMEM((tpt,), jnp.float32),
            pltpu.VMEM((tpt,), jnp.float32),
            pltpu.VMEM((S,), jnp.int32),
            pltpu.VMEM((S,), jnp.int32),
            pltpu.VMEM((slots,), jnp.int32),
            pltpu.VMEM((rows_chunk, M), jnp.float32),
            pltpu.VMEM((rows_chunk, M), jnp.float32),
            pltpu.VMEM_SHARED((NS, L), jnp.int32),
            pltpu.VMEM_SHARED((S,), jnp.int32),
            pltpu.VMEM_SHARED((S,), jnp.int32),
            pltpu.SemaphoreType.DMA,
            pltpu.SemaphoreType.DMA,
            pltpu.SemaphoreType.DMA,
            pltpu.SemaphoreType.DMA,
        ],
    )


def _expert_body(disp_ref, We_ref, be_ref, eo_ref):
    acc = lax.dot_general(disp_ref[...].astype(jnp.bfloat16),
                          We_ref[0].astype(jnp.bfloat16),
                          (((1,), (1,)), ((), ())),
                          preferred_element_type=jnp.float32)
    eo_ref[...] = acc + be_ref[0]


def _make_combine(S, C, E, Dout):
    per_w = S // NW
    tchunk = 16
    nch = per_w // tchunk

    def body(eo_ref, dst1_ref, dst2_ref, w1_ref, w2_ref, y_ref,
             d1_v, d2_v, w1_v, w2_v,
             r1a, r1b, r2a, r2b, outa, outb,
             sg1a, sg1b, sg2a, sg2b, swa, swb):
        wid = lax.axis_index("s") * NC + lax.axis_index("c")
        base = wid * per_w
        pltpu.sync_copy(dst1_ref.at[pl.ds(base, per_w)], d1_v)
        pltpu.sync_copy(dst2_ref.at[pl.ds(base, per_w)], d2_v)
        pltpu.sync_copy(w1_ref.at[pl.ds(base, per_w)], w1_v)
        pltpu.sync_copy(w2_ref.at[pl.ds(base, per_w)], w2_v)
        r1s = [r1a, r1b]
        r2s = [r2a, r2b]
        outs = [outa, outb]
        sg1 = [sg1a, sg1b]
        sg2 = [sg2a, sg2b]
        sw = [swa, swb]

        def gath(h):
            b = h % 2
            c1 = pltpu.async_copy(
                eo_ref.at[d1_v.at[pl.ds(h * tchunk, tchunk)]], r1s[b], sg1[b])
            c2 = pltpu.async_copy(
                eo_ref.at[d2_v.at[pl.ds(h * tchunk, tchunk)]], r2s[b], sg2[b])
            return (c1, c2)

        gs = [gath(0), gath(1)]
        ws = [None, None]
        for h in range(nch):
            b = h % 2
            gs[b][0].wait()
            gs[b][1].wait()
            if h >= 2:
                ws[b].wait()

            def tok(t, carry):
                widx = jnp.broadcast_to(h * tchunk + t, (L,)).astype(jnp.int32)
                w1b = plsc.load_gather(w1_v, [widx])
                w2b = plsc.load_gather(w2_v, [widx])
                for k in range(Dout // L):
                    dsk = pl.ds(k * L, L)
                    outs[b][t, dsk] = (r1s[b][t, dsk] * w1b +
                                       r2s[b][t, dsk] * w2b)
                return carry

            lax.fori_loop(0, tchunk, tok, 0)
            ws[b] = pltpu.async_copy(
                outs[b], y_ref.at[pl.ds(base + h * tchunk, tchunk)], sw[b])
            if h + 2 < nch:
                gs[b] = gath(h + 2)
        ws[(nch - 2) % 2].wait()
        ws[(nch - 1) % 2].wait()

    mesh = plsc.VectorSubcoreMesh(core_axis_name="c", subcore_axis_name="s")
    return pl.kernel(
        body,
        out_type=jax.ShapeDtypeStruct((S, Dout), jnp.float32),
        mesh=mesh,
        compiler_params=pltpu.CompilerParams(needs_layout_passes=False),
        scratch_types=[
            pltpu.VMEM((per_w,), jnp.int32),
            pltpu.VMEM((per_w,), jnp.int32),
            pltpu.VMEM((per_w,), jnp.float32),
            pltpu.VMEM((per_w,), jnp.float32),
            pltpu.VMEM((tchunk, Dout), jnp.float32),
            pltpu.VMEM((tchunk, Dout), jnp.float32),
            pltpu.VMEM((tchunk, Dout), jnp.float32),
            pltpu.VMEM((tchunk, Dout), jnp.float32),
            pltpu.VMEM((tchunk, Dout), jnp.float32),
            pltpu.VMEM((tchunk, Dout), jnp.float32),
            pltpu.SemaphoreType.DMA,
            pltpu.SemaphoreType.DMA,
            pltpu.SemaphoreType.DMA,
            pltpu.SemaphoreType.DMA,
            pltpu.SemaphoreType.DMA,
            pltpu.SemaphoreType.DMA,
        ],
    )


@jax.jit
def kernel(x, wg, We, be, gnoise):
    seq, tok, M = x.shape
    S = seq * tok
    E = wg.shape[0]
    Dout = We.shape[1]
    C = 2 * S // E

    xf = x.reshape(S, M)
    gnT = gnoise.T

    logT, noisyT = pl.pallas_call(
        _logits_body,
        out_shape=[jax.ShapeDtypeStruct((E, S), jnp.float32),
                   jax.ShapeDtypeStruct((E, S), jnp.float32)],
    )(xf, wg, gnT)

    disp, dst1, dst2, w1, w2 = _make_gate_dispatch(E, S, C, M)(
        logT, noisyT, xf)

    eo = pl.pallas_call(
        _expert_body,
        grid=(E,),
        in_specs=[
            pl.BlockSpec((C, M), lambda e: (e, 0)),
            pl.BlockSpec((1, Dout, M), lambda e: (e, 0, 0)),
            pl.BlockSpec((1, 1, Dout), lambda e: (e, 0, 0)),
        ],
        out_specs=pl.BlockSpec((C, Dout), lambda e: (e, 0)),
        out_shape=jax.ShapeDtypeStruct((E * C, Dout), jnp.float32),
    )(disp, We, be.reshape(E, 1, Dout))

    y = _make_combine(S, C, E, Dout)(eo, dst1, dst2, w1, w2)
    return y.reshape(x.shape)


# speedup vs baseline: 1.3811x; 1.0088x over previous
import jax
import jax.numpy as jnp
from jax import lax
from jax.experimental import pallas as pl
from jax.experimental.pallas import tpu as pltpu
from jax.experimental.pallas import tpu_sc as plsc

NC, NS, L = 2, 16, 16
NW = NC * NS
F32_EPS = 1.1920929e-07


def _logits_body(x_ref, wg_ref, gnT_ref, logT_ref, noisyT_ref):
    lg = lax.dot_general(wg_ref[...], x_ref[...], (((1,), (1,)), ((), ())),
                         preferred_element_type=jnp.float32)
    logT_ref[...] = lg
    noisyT_ref[...] = lg + gnT_ref[...]


def _make_gate_dispatch(E, S, C, M):
    slots = E * C
    tpt = S // NS
    spw = slots // NW
    n_loc = tpt // L
    rows_chunk = 48

    def body(logT, noisyT, x_ref,
             disp_ref, dst1_hbm, dst2_hbm, w1_hbm, w2_hbm,
             lg_v, ns_v, e1l, e2l, g1l, g2l, r1l, r2l,
             cnts_v, allc_v, tot_v, off_v,
             d1m_v, d2m_v, d1o_v, d2o_v, w1o_v, w2o_v,
             d1a_v, d2a_v, src_v, rows_v, rows2_v,
             cnts_sh, d1_sh, d2_sh, semr0, semr1, semw0, semw1):
        cid = lax.axis_index("c")
        sid = lax.axis_index("s")
        wid = sid * NC + cid
        s0 = sid * tpt
        iota = lax.iota(jnp.int32, L)

        pltpu.sync_copy(logT.at[:, pl.ds(s0, tpt)], lg_v)
        pltpu.sync_copy(noisyT.at[:, pl.ds(s0, tpt)], ns_v)

        c1 = [jnp.int32(0)] * E
        c2 = [jnp.int32(0)] * E
        for c in range(n_loc):
            ds = pl.ds(c * L, L)
            dsl = ds
            ls = [lg_v[e, ds] for e in range(E)]
            m = ls[0]
            for e in range(1, E):
                m = jnp.maximum(m, ls[e])
            exs = [jnp.exp(ls[e] - m) for e in range(E)]
            Z = exs[0]
            for e in range(1, E):
                Z = Z + exs[e]
            idx1 = jnp.full((L,), 127, jnp.int32)
            for e in range(E):
                idx1 = jnp.minimum(idx1, jnp.where(ls[e] == m, e, 127))
            g1 = jnp.zeros((L,), jnp.float32)
            for e in range(E):
                g1 = jnp.where(idx1 == e, exs[e], g1)
            neg = jnp.full((L,), -jnp.inf, jnp.float32)
            nss = []
            m2 = neg
            for e in range(E):
                nv = jnp.where(idx1 == e, neg, ns_v[e, ds])
                nss.append(nv)
                m2 = jnp.maximum(m2, nv)
            idx2 = jnp.full((L,), 127, jnp.int32)
            for e in range(E):
                idx2 = jnp.minimum(idx2, jnp.where(nss[e] == m2, e, 127))
            g2 = jnp.zeros((L,), jnp.float32)
            for e in range(E):
                g2 = jnp.where(idx2 == e, exs[e], g2)
            rZ = 1.0 / Z
            e1l[dsl] = idx1
            e2l[dsl] = idx2
            g1l[dsl] = g1 * rZ
            g2l[dsl] = g2 * rZ
            r1 = jnp.zeros((L,), jnp.int32)
            r2 = jnp.zeros((L,), jnp.int32)
            for e in range(E):
                mask = idx1 == e
                mi = jnp.where(mask, 1, 0).astype(jnp.int32)
                pc = plsc.cumsum(mi)
                r1 = jnp.where(mask, c1[e] + pc - 1, r1)
                c1[e] = c1[e] + jnp.sum(mi)
                mask = idx2 == e
                mi = jnp.where(mask, 1, 0).astype(jnp.int32)
                pc = plsc.cumsum(mi)
                r2 = jnp.where(mask, c2[e] + pc - 1, r2)
                c2[e] = c2[e] + jnp.sum(mi)
            r1l[dsl] = r1
            r2l[dsl] = r2

        cv = jnp.zeros((L,), jnp.int32)
        for e in range(E):
            cv = jnp.where(iota == e, c1[e], cv)
            cv = jnp.where(iota == (E + e), c2[e], cv)
        cnts_v[...] = cv
        pltpu.sync_copy(cnts_v, cnts_sh.at[sid])
        plsc.subcore_barrier()

        pltpu.sync_copy(cnts_sh, allc_v)
        sv = jnp.broadcast_to(sid, (L,)).astype(jnp.int32)
        pref = jnp.zeros((L,), jnp.int32)
        tot = jnp.zeros((L,), jnp.int32)
        for t in range(NS):
            row = allc_v[t]
            tv = jnp.full((L,), t, jnp.int32)
            pref = pref + jnp.where(tv < sv, row, 0)
            tot = tot + row
        tot_v[...] = tot
        sh = plsc.load_gather(tot_v, [jnp.maximum(iota - E, 0)])
        off = pref + jnp.where(iota >= E, sh, 0)
        off_v[...] = off

        for c in range(n_loc):
            dsl = pl.ds(c * L, L)
            idx1 = e1l[dsl]
            idx2 = e2l[dsl]
            base1 = plsc.load_gather(off_v, [idx1])
            base2 = plsc.load_gather(off_v, [idx2 + E])
            l1 = base1 + r1l[dsl]
            l2 = base2 + r2l[dsl]
            k1 = l1 < C
            k2 = l2 < C
            g1k = jnp.where(k1, g1l[dsl], 0.0)
            g2k = jnp.where(k2, g2l[dsl], 0.0)
            rd = 1.0 / jnp.maximum(g1k + g2k, F32_EPS)
            d1 = idx1 * C + l1
            d2 = idx2 * C + l2
            d1m_v[dsl] = jnp.where(k1, d1, slots)
            d2m_v[dsl] = jnp.where(k2, d2, slots)
            d1o_v[dsl] = jnp.where(k1, d1, 0)
            d2o_v[dsl] = jnp.where(k2, d2, 0)
            w1o_v[dsl] = g1k * rd
            w2o_v[dsl] = g2k * rd

        @pl.when(cid == 0)
        def _():
            pltpu.sync_copy(d1o_v, dst1_hbm.at[pl.ds(s0, tpt)])
            pltpu.sync_copy(d2o_v, dst2_hbm.at[pl.ds(s0, tpt)])
            pltpu.sync_copy(w1o_v, w1_hbm.at[pl.ds(s0, tpt)])
            pltpu.sync_copy(w2o_v, w2_hbm.at[pl.ds(s0, tpt)])

        pltpu.sync_copy(d1m_v, d1_sh.at[pl.ds(s0, tpt)])
        pltpu.sync_copy(d2m_v, d2_sh.at[pl.ds(s0, tpt)])
        plsc.subcore_barrier()

        pltpu.sync_copy(d1_sh, d1a_v)
        pltpu.sync_copy(d2_sh, d2a_v)

        def zinit(i, carry):
            src_v[pl.ds(pl.multiple_of(i * L, L), L)] = jnp.zeros(
                (L,), jnp.int32)
            return carry

        lax.fori_loop(0, slots // L, zinit, 0)

        def rebuild(i, carry):
            ds = pl.ds(pl.multiple_of(i * L, L), L)
            tid = (i * L + iota).astype(jnp.int32)
            d1c = d1a_v[ds]
            plsc.store_scatter(src_v, [d1c], tid, mask=d1c < slots)
            d2c = d2a_v[ds]
            plsc.store_scatter(src_v, [d2c], tid, mask=d2c < slots)
            return carry

        lax.fori_loop(0, S // L, rebuild, 0)

        base = wid * spw
        offs = [0, 48, 96]
        szs = [48, 48, 32]
        nchunks = len(offs)
        bufs = [rows_v, rows2_v]
        rsems = [semr0, semr1]
        wsems = [semw0, semw1]

        def rd(j):
            return pltpu.async_copy(
                x_ref.at[src_v.at[pl.ds(base + offs[j], szs[j])]],
                bufs[j % 2].at[pl.ds(0, szs[j])], rsems[j % 2])

        def wr(j):
            return pltpu.async_copy(
                bufs[j % 2].at[pl.ds(0, szs[j])],
                disp_ref.at[pl.ds(base + offs[j], szs[j])],
                wsems[j % 2])

        rds = [rd(0), rd(1)]
        wrs = [None, None]
        for j in range(nchunks):
            rds[j % 2].wait()
            wrs[j % 2] = wr(j)
            if j + 2 < nchunks:
                wrs[j % 2].wait()
                rds[j % 2] = rd(j + 2)
        wrs[(nchunks - 2) % 2].wait()
        wrs[(nchunks - 1) % 2].wait()

    mesh = plsc.VectorSubcoreMesh(core_axis_name="c", subcore_axis_name="s")
    return pl.kernel(
        body,
        out_type=[
            jax.ShapeDtypeStruct((slots, M), jnp.float32),
            jax.ShapeDtypeStruct((S,), jnp.int32),
            jax.ShapeDtypeStruct((S,), jnp.int32),
            jax.ShapeDtypeStruct((S,), jnp.float32),
            jax.ShapeDtypeStruct((S,), jnp.float32),
        ],
        mesh=mesh,
        compiler_params=pltpu.CompilerParams(needs_layout_passes=False),
        scratch_types=[
            pltpu.VMEM((E, tpt), jnp.float32),
            pltpu.VMEM((E, tpt), jnp.float32),
            pltpu.VMEM((tpt,), jnp.int32),
            pltpu.VMEM((tpt,), jnp.int32),
            pltpu.VMEM((tpt,), jnp.float32),
            pltpu.VMEM((tpt,), jnp.float32),
            pltpu.VMEM((tpt,), jnp.int32),
            pltpu.VMEM((tpt,), jnp.int32),
            pltpu.VMEM((L,), jnp.int32),
            pltpu.VMEM((NS, L), jnp.int32),
            pltpu.VMEM((L,), jnp.int32),
            pltpu.VMEM((L,), jnp.int32),
            pltpu.VMEM((tpt,), jnp.int32),
            pltpu.VMEM((tpt,), jnp.int32),
            pltpu.VMEM((tpt,), jnp.int32),
            pltpu.VMEM((tpt,), jnp.int32),
            pltpu.VMEM((tpt,), jnp.float32),
            pltpu.VMEM((tpt,), jnp.float32),
            pltpu.VMEM((S,), jnp.int32),
            pltpu.VMEM((S,), jnp.int32),
            pltpu.VMEM((slots,), jnp.int32),
            pltpu.VMEM((rows_chunk, M), jnp.float32),
            pltpu.VMEM((rows_chunk, M), jnp.float32),
            pltpu.VMEM_SHARED((NS, L), jnp.int32),
            pltpu.VMEM_SHARED((S,), jnp.int32),
            pltpu.VMEM_SHARED((S,), jnp.int32),
            pltpu.SemaphoreType.DMA,
            pltpu.SemaphoreType.DMA,
            pltpu.SemaphoreType.DMA,
            pltpu.SemaphoreType.DMA,
        ],
    )


def _expert_body(disp_ref, We_ref, be_ref, eo_ref):
    acc = lax.dot_general(disp_ref[...].astype(jnp.bfloat16),
                          We_ref[0].astype(jnp.bfloat16),
                          (((1,), (1,)), ((), ())),
                          preferred_element_type=jnp.float32)
    eo_ref[...] = acc + be_ref[0]


def _make_combine(S, C, E, Dout):
    per_w = S // NW
    tchunk = 16
    nch = per_w // tchunk

    def body(eo_ref, dst1_ref, dst2_ref, w1_ref, w2_ref, y_ref,
             d1_v, d2_v, w1_v, w2_v,
             r1a, r1b, r2a, r2b, outa, outb,
             sg1a, sg1b, sg2a, sg2b, swa, swb):
        wid = lax.axis_index("s") * NC + lax.axis_index("c")
        base = wid * per_w
        pltpu.sync_copy(dst1_ref.at[pl.ds(base, per_w)], d1_v)
        pltpu.sync_copy(dst2_ref.at[pl.ds(base, per_w)], d2_v)
        pltpu.sync_copy(w1_ref.at[pl.ds(base, per_w)], w1_v)
        pltpu.sync_copy(w2_ref.at[pl.ds(base, per_w)], w2_v)
        r1s = [r1a, r1b]
        r2s = [r2a, r2b]
        outs = [outa, outb]
        sg1 = [sg1a, sg1b]
        sg2 = [sg2a, sg2b]
        sw = [swa, swb]

        def gath(h):
            b = h % 2
            c1 = pltpu.async_copy(
                eo_ref.at[d1_v.at[pl.ds(h * tchunk, tchunk)]], r1s[b], sg1[b])
            c2 = pltpu.async_copy(
                eo_ref.at[d2_v.at[pl.ds(h * tchunk, tchunk)]], r2s[b], sg2[b])
            return (c1, c2)

        gs = [gath(0), gath(1)]
        ws = [None, None]
        for h in range(nch):
            b = h % 2
            gs[b][0].wait()
            gs[b][1].wait()
            if h >= 2:
                ws[b].wait()

            def tok(t, carry):
                widx = jnp.broadcast_to(h * tchunk + t, (L,)).astype(jnp.int32)
                w1b = plsc.load_gather(w1_v, [widx])
                w2b = plsc.load_gather(w2_v, [widx])
                for k in range(Dout // L):
                    dsk = pl.ds(k * L, L)
                    outs[b][t, dsk] = (r1s[b][t, dsk] * w1b +
                                       r2s[b][t, dsk] * w2b)
                return carry

            lax.fori_loop(0, tchunk, tok, 0)
            ws[b] = pltpu.async_copy(
                outs[b], y_ref.at[pl.ds(base + h * tchunk, tchunk)], sw[b])
            if h + 2 < nch:
                gs[b] = gath(h + 2)
        ws[(nch - 2) % 2].wait()
        ws[(nch - 1) % 2].wait()

    mesh = plsc.VectorSubcoreMesh(core_axis_name="c", subcore_axis_name="s")
    return pl.kernel(
        body,
        out_type=jax.ShapeDtypeStruct((S, Dout), jnp.float32),
        mesh=mesh,
        compiler_params=pltpu.CompilerParams(needs_layout_passes=False),
        scratch_types=[
            pltpu.VMEM((per_w,), jnp.int32),
            pltpu.VMEM((per_w,), jnp.int32),
            pltpu.VMEM((per_w,), jnp.float32),
            pltpu.VMEM((per_w,), jnp.float32),
            pltpu.VMEM((tchunk, Dout), jnp.float32),
            pltpu.VMEM((tchunk, Dout), jnp.float32),
            pltpu.VMEM((tchunk, Dout), jnp.float32),
            pltpu.VMEM((tchunk, Dout), jnp.float32),
            pltpu.VMEM((tchunk, Dout), jnp.float32),
            pltpu.VMEM((tchunk, Dout), jnp.float32),
            pltpu.SemaphoreType.DMA,
            pltpu.SemaphoreType.DMA,
            pltpu.SemaphoreType.DMA,
            pltpu.SemaphoreType.DMA,
            pltpu.SemaphoreType.DMA,
            pltpu.SemaphoreType.DMA,
        ],
    )


@jax.jit
def kernel(x, wg, We, be, gnoise):
    seq, tok, M = x.shape
    S = seq * tok
    E = wg.shape[0]
    Dout = We.shape[1]
    C = 2 * S // E

    xf = x.reshape(S, M)
    gnT = gnoise.T

    logT, noisyT = pl.pallas_call(
        _logits_body,
        out_shape=[jax.ShapeDtypeStruct((E, S), jnp.float32),
                   jax.ShapeDtypeStruct((E, S), jnp.float32)],
    )(xf, wg, gnT)

    disp, dst1, dst2, w1, w2 = _make_gate_dispatch(E, S, C, M)(
        logT, noisyT, xf)

    eo = pl.pallas_call(
        _expert_body,
        grid=(E,),
        in_specs=[
            pl.BlockSpec((C, M), lambda e: (e, 0)),
            pl.BlockSpec((1, Dout, M), lambda e: (e, 0, 0)),
            pl.BlockSpec((1, 1, Dout), lambda e: (e, 0, 0)),
        ],
        out_specs=pl.BlockSpec((C, Dout), lambda e: (e, 0)),
        out_shape=jax.ShapeDtypeStruct((E * C, Dout), jnp.float32),
    )(disp, We, be.reshape(E, 1, Dout))

    y = _make_combine(S, C, E, Dout)(eo, dst1, dst2, w1, w2)
    return y.reshape(x.shape)


# speedup vs baseline: 1.4020x; 1.0151x over previous
import jax
import jax.numpy as jnp
from jax import lax
from jax.experimental import pallas as pl
from jax.experimental.pallas import tpu as pltpu
from jax.experimental.pallas import tpu_sc as plsc

NC, NS, L = 2, 16, 16
NW = NC * NS
F32_EPS = 1.1920929e-07


def _logits_body(x_ref, wg_ref, gnT_ref, logT_ref, noisyT_ref):
    lg = lax.dot_general(wg_ref[...], x_ref[...], (((1,), (1,)), ((), ())),
                         preferred_element_type=jnp.float32)
    logT_ref[...] = lg
    noisyT_ref[...] = lg + gnT_ref[...]


def _make_gate_dispatch(E, S, C, M):
    slots = E * C
    tpt = S // NS
    spw = slots // NW
    n_loc = tpt // L
    rows_chunk = 48

    def body(logT, noisyT, x_ref,
             disp_ref, dst1_hbm, dst2_hbm, w1_hbm, w2_hbm,
             lg_v, ns_v, e1l, e2l, g1l, g2l, r1l, r2l,
             cnts_v, allc_v, tot_v, off_v,
             d1m_v, d2m_v, d1o_v, d2o_v, w1o_v, w2o_v,
             d1a_v, d2a_v, src_v, rows_v, rows2_v,
             cnts_sh, d1_sh, d2_sh, semr0, semr1, semw0, semw1):
        cid = lax.axis_index("c")
        sid = lax.axis_index("s")
        wid = sid * NC + cid
        s0 = sid * tpt
        iota = lax.iota(jnp.int32, L)

        pltpu.sync_copy(logT.at[:, pl.ds(s0, tpt)], lg_v)
        pltpu.sync_copy(noisyT.at[:, pl.ds(s0, tpt)], ns_v)

        c1 = [jnp.int32(0)] * E
        c2 = [jnp.int32(0)] * E
        for c in range(n_loc):
            ds = pl.ds(c * L, L)
            dsl = ds
            ls = [lg_v[e, ds] for e in range(E)]
            m = ls[0]
            for e in range(1, E):
                m = jnp.maximum(m, ls[e])
            exs = [jnp.exp(ls[e] - m) for e in range(E)]
            Z = exs[0]
            for e in range(1, E):
                Z = Z + exs[e]
            idx1 = jnp.full((L,), 127, jnp.int32)
            for e in range(E):
                idx1 = jnp.minimum(idx1, jnp.where(ls[e] == m, e, 127))
            g1 = jnp.zeros((L,), jnp.float32)
            for e in range(E):
                g1 = jnp.where(idx1 == e, exs[e], g1)
            neg = jnp.full((L,), -jnp.inf, jnp.float32)
            nss = []
            m2 = neg
            for e in range(E):
                nv = jnp.where(idx1 == e, neg, ns_v[e, ds])
                nss.append(nv)
                m2 = jnp.maximum(m2, nv)
            idx2 = jnp.full((L,), 127, jnp.int32)
            for e in range(E):
                idx2 = jnp.minimum(idx2, jnp.where(nss[e] == m2, e, 127))
            g2 = jnp.zeros((L,), jnp.float32)
            for e in range(E):
                g2 = jnp.where(idx2 == e, exs[e], g2)
            rZ = 1.0 / Z
            e1l[dsl] = idx1
            e2l[dsl] = idx2
            g1l[dsl] = g1 * rZ
            g2l[dsl] = g2 * rZ
            r1 = jnp.zeros((L,), jnp.int32)
            r2 = jnp.zeros((L,), jnp.int32)
            for e in range(E):
                mask = idx1 == e
                mi = jnp.where(mask, 1, 0).astype(jnp.int32)
                pc = plsc.cumsum(mi)
                r1 = jnp.where(mask, c1[e] + pc - 1, r1)
                c1[e] = c1[e] + jnp.sum(mi)
                mask = idx2 == e
                mi = jnp.where(mask, 1, 0).astype(jnp.int32)
                pc = plsc.cumsum(mi)
                r2 = jnp.where(mask, c2[e] + pc - 1, r2)
                c2[e] = c2[e] + jnp.sum(mi)
            r1l[dsl] = r1
            r2l[dsl] = r2

        cv = jnp.zeros((L,), jnp.int32)
        for e in range(E):
            cv = jnp.where(iota == e, c1[e], cv)
            cv = jnp.where(iota == (E + e), c2[e], cv)
        cnts_v[...] = cv
        pltpu.sync_copy(cnts_v, cnts_sh.at[sid])
        plsc.subcore_barrier()

        pltpu.sync_copy(cnts_sh, allc_v)
        sv = jnp.broadcast_to(sid, (L,)).astype(jnp.int32)
        pref = jnp.zeros((L,), jnp.int32)
        tot = jnp.zeros((L,), jnp.int32)
        for t in range(NS):
            row = allc_v[t]
            tv = jnp.full((L,), t, jnp.int32)
            pref = pref + jnp.where(tv < sv, row, 0)
            tot = tot + row
        tot_v[...] = tot
        sh = plsc.load_gather(tot_v, [jnp.maximum(iota - E, 0)])
        off = pref + jnp.where(iota >= E, sh, 0)
        off_v[...] = off

        for c in range(n_loc):
            dsl = pl.ds(c * L, L)
            idx1 = e1l[dsl]
            idx2 = e2l[dsl]
            base1 = plsc.load_gather(off_v, [idx1])
            base2 = plsc.load_gather(off_v, [idx2 + E])
            l1 = base1 + r1l[dsl]
            l2 = base2 + r2l[dsl]
            k1 = l1 < C
            k2 = l2 < C
            g1k = jnp.where(k1, g1l[dsl], 0.0)
            g2k = jnp.where(k2, g2l[dsl], 0.0)
            rd = 1.0 / jnp.maximum(g1k + g2k, F32_EPS)
            d1 = idx1 * C + l1
            d2 = idx2 * C + l2
            d1m_v[dsl] = jnp.where(k1, d1, slots)
            d2m_v[dsl] = jnp.where(k2, d2, slots)
            d1o_v[dsl] = jnp.where(k1, d1, 0)
            d2o_v[dsl] = jnp.where(k2, d2, 0)
            w1o_v[dsl] = g1k * rd
            w2o_v[dsl] = g2k * rd

        @pl.when(cid == 0)
        def _():
            pltpu.sync_copy(d1o_v, dst1_hbm.at[pl.ds(s0, tpt)])
            pltpu.sync_copy(d2o_v, dst2_hbm.at[pl.ds(s0, tpt)])
            pltpu.sync_copy(w1o_v, w1_hbm.at[pl.ds(s0, tpt)])
            pltpu.sync_copy(w2o_v, w2_hbm.at[pl.ds(s0, tpt)])

        pltpu.sync_copy(d1m_v, d1_sh.at[pl.ds(s0, tpt)])
        pltpu.sync_copy(d2m_v, d2_sh.at[pl.ds(s0, tpt)])
        plsc.subcore_barrier()

        pltpu.sync_copy(d1_sh, d1a_v)
        pltpu.sync_copy(d2_sh, d2a_v)

        def zinit(i, carry):
            src_v[pl.ds(pl.multiple_of(i * L, L), L)] = jnp.zeros(
                (L,), jnp.int32)
            return carry

        lax.fori_loop(0, slots // L, zinit, 0)

        def rebuild(i, carry):
            ds = pl.ds(pl.multiple_of(i * L, L), L)
            tid = (i * L + iota).astype(jnp.int32)
            d1c = d1a_v[ds]
            plsc.store_scatter(src_v, [d1c], tid, mask=d1c < slots)
            d2c = d2a_v[ds]
            plsc.store_scatter(src_v, [d2c], tid, mask=d2c < slots)
            return carry

        lax.fori_loop(0, S // L, rebuild, 0)

        base = wid * spw
        offs = [0, 48, 96]
        szs = [48, 48, 32]
        nchunks = len(offs)
        bufs = [rows_v, rows2_v]
        rsems = [semr0, semr1]
        wsems = [semw0, semw1]

        def rd(j):
            return pltpu.async_copy(
                x_ref.at[src_v.at[pl.ds(base + offs[j], szs[j])]],
                bufs[j % 2].at[pl.ds(0, szs[j])], rsems[j % 2])

        def wr(j):
            return pltpu.async_copy(
                bufs[j % 2].at[pl.ds(0, szs[j])],
                disp_ref.at[pl.ds(base + offs[j], szs[j])],
                wsems[j % 2])

        rds = [rd(0), rd(1)]
        wrs = [None, None]
        for j in range(nchunks):
            rds[j % 2].wait()
            wrs[j % 2] = wr(j)
            if j + 2 < nchunks:
                wrs[j % 2].wait()
                rds[j % 2] = rd(j + 2)
        wrs[(nchunks - 2) % 2].wait()
        wrs[(nchunks - 1) % 2].wait()

    mesh = plsc.VectorSubcoreMesh(core_axis_name="c", subcore_axis_name="s")
    return pl.kernel(
        body,
        out_type=[
            jax.ShapeDtypeStruct((slots, M), jnp.float32),
            jax.ShapeDtypeStruct((S,), jnp.int32),
            jax.ShapeDtypeStruct((S,), jnp.int32),
            jax.ShapeDtypeStruct((S,), jnp.float32),
            jax.ShapeDtypeStruct((S,), jnp.float32),
        ],
        mesh=mesh,
        compiler_params=pltpu.CompilerParams(needs_layout_passes=False),
        scratch_types=[
            pltpu.VMEM((E, tpt), jnp.float32),
            pltpu.VMEM((E, tpt), jnp.float32),
            pltpu.VMEM((tpt,), jnp.int32),
            pltpu.VMEM((tpt,), jnp.int32),
            pltpu.VMEM((tpt,), jnp.float32),
            pltpu.VMEM((tpt,), jnp.float32),
            pltpu.VMEM((tpt,), jnp.int32),
            pltpu.VMEM((tpt,), jnp.int32),
            pltpu.VMEM((L,), jnp.int32),
            pltpu.VMEM((NS, L), jnp.int32),
            pltpu.VMEM((L,), jnp.int32),
            pltpu.VMEM((L,), jnp.int32),
            pltpu.VMEM((tpt,), jnp.int32),
            pltpu.VMEM((tpt,), jnp.int32),
            pltpu.VMEM((tpt,), jnp.int32),
            pltpu.VMEM((tpt,), jnp.int32),
            pltpu.VMEM((tpt,), jnp.float32),
            pltpu.VMEM((tpt,), jnp.float32),
            pltpu.VMEM((S,), jnp.int32),
            pltpu.VMEM((S,), jnp.int32),
            pltpu.VMEM((slots,), jnp.int32),
            pltpu.VMEM((rows_chunk, M), jnp.float32),
            pltpu.VMEM((rows_chunk, M), jnp.float32),
            pltpu.VMEM_SHARED((NS, L), jnp.int32),
            pltpu.VMEM_SHARED((S,), jnp.int32),
            pltpu.VMEM_SHARED((S,), jnp.int32),
            pltpu.SemaphoreType.DMA,
            pltpu.SemaphoreType.DMA,
            pltpu.SemaphoreType.DMA,
            pltpu.SemaphoreType.DMA,
        ],
    )


def _expert_body(disp_ref, We_ref, be_ref, eo_ref):
    acc = lax.dot_general(disp_ref[...].astype(jnp.bfloat16),
                          We_ref[0].astype(jnp.bfloat16),
                          (((1,), (1,)), ((), ())),
                          preferred_element_type=jnp.float32)
    eo_ref[...] = acc + be_ref[0]


def _make_combine(S, C, E, Dout):
    per_w = S // NW
    tchunk = 16
    nch = per_w // tchunk

    def body(eo_ref, dst1_ref, dst2_ref, w1_ref, w2_ref, y_ref,
             d1_v, d2_v, w1_v, w2_v,
             r1a, r1b, r2a, r2b, outa, outb,
             sg1a, sg1b, sg2a, sg2b, swa, swb):
        wid = lax.axis_index("s") * NC + lax.axis_index("c")
        base = wid * per_w
        m1 = pltpu.async_copy(dst1_ref.at[pl.ds(base, per_w)], d1_v, sg1a)
        m2 = pltpu.async_copy(dst2_ref.at[pl.ds(base, per_w)], d2_v, sg2a)
        m3 = pltpu.async_copy(w1_ref.at[pl.ds(base, per_w)], w1_v, sg1b)
        m4 = pltpu.async_copy(w2_ref.at[pl.ds(base, per_w)], w2_v, sg2b)
        m1.wait()
        m2.wait()
        m3.wait()
        m4.wait()
        r1s = [r1a, r1b]
        r2s = [r2a, r2b]
        outs = [outa, outb]
        sg1 = [sg1a, sg1b]
        sg2 = [sg2a, sg2b]
        sw = [swa, swb]

        def gath(h):
            b = h % 2
            c1 = pltpu.async_copy(
                eo_ref.at[d1_v.at[pl.ds(h * tchunk, tchunk)]], r1s[b], sg1[b])
            c2 = pltpu.async_copy(
                eo_ref.at[d2_v.at[pl.ds(h * tchunk, tchunk)]], r2s[b], sg2[b])
            return (c1, c2)

        gs = [gath(0), gath(1)]
        ws = [None, None]
        for h in range(nch):
            b = h % 2
            gs[b][0].wait()
            gs[b][1].wait()
            if h >= 2:
                ws[b].wait()

            def tok(t, carry):
                widx = jnp.broadcast_to(h * tchunk + t, (L,)).astype(jnp.int32)
                w1b = plsc.load_gather(w1_v, [widx])
                w2b = plsc.load_gather(w2_v, [widx])
                for k in range(Dout // L):
                    dsk = pl.ds(k * L, L)
                    outs[b][t, dsk] = (r1s[b][t, dsk] * w1b +
                                       r2s[b][t, dsk] * w2b)
                return carry

            lax.fori_loop(0, tchunk, tok, 0)
            ws[b] = pltpu.async_copy(
                outs[b], y_ref.at[pl.ds(base + h * tchunk, tchunk)], sw[b])
            if h + 2 < nch:
                gs[b] = gath(h + 2)
        ws[(nch - 2) % 2].wait()
        ws[(nch - 1) % 2].wait()

    mesh = plsc.VectorSubcoreMesh(core_axis_name="c", subcore_axis_name="s")
    return pl.kernel(
        body,
        out_type=jax.ShapeDtypeStruct((S, Dout), jnp.float32),
        mesh=mesh,
        compiler_params=pltpu.CompilerParams(needs_layout_passes=False),
        scratch_types=[
            pltpu.VMEM((per_w,), jnp.int32),
            pltpu.VMEM((per_w,), jnp.int32),
            pltpu.VMEM((per_w,), jnp.float32),
            pltpu.VMEM((per_w,), jnp.float32),
            pltpu.VMEM((tchunk, Dout), jnp.float32),
            pltpu.VMEM((tchunk, Dout), jnp.float32),
            pltpu.VMEM((tchunk, Dout), jnp.float32),
            pltpu.VMEM((tchunk, Dout), jnp.float32),
            pltpu.VMEM((tchunk, Dout), jnp.float32),
            pltpu.VMEM((tchunk, Dout), jnp.float32),
            pltpu.SemaphoreType.DMA,
            pltpu.SemaphoreType.DMA,
            pltpu.SemaphoreType.DMA,
            pltpu.SemaphoreType.DMA,
            pltpu.SemaphoreType.DMA,
            pltpu.SemaphoreType.DMA,
        ],
    )


@jax.jit
def kernel(x, wg, We, be, gnoise):
    seq, tok, M = x.shape
    S = seq * tok
    E = wg.shape[0]
    Dout = We.shape[1]
    C = 2 * S // E

    xf = x.reshape(S, M)
    gnT = gnoise.T

    logT, noisyT = pl.pallas_call(
        _logits_body,
        out_shape=[jax.ShapeDtypeStruct((E, S), jnp.float32),
                   jax.ShapeDtypeStruct((E, S), jnp.float32)],
    )(xf, wg, gnT)

    disp, dst1, dst2, w1, w2 = _make_gate_dispatch(E, S, C, M)(
        logT, noisyT, xf)

    eo = pl.pallas_call(
        _expert_body,
        grid=(E,),
        in_specs=[
            pl.BlockSpec((C, M), lambda e: (e, 0)),
            pl.BlockSpec((1, Dout, M), lambda e: (e, 0, 0)),
            pl.BlockSpec((1, 1, Dout), lambda e: (e, 0, 0)),
        ],
        out_specs=pl.BlockSpec((C, Dout), lambda e: (e, 0)),
        out_shape=jax.ShapeDtypeStruct((E * C, Dout), jnp.float32),
    )(disp, We, be.reshape(E, 1, Dout))

    y = _make_combine(S, C, E, Dout)(eo, dst1, dst2, w1, w2)
    return y.reshape(x.shape)
